# Initial kernel scaffold; baseline (speedup 1.0000x reference)
import jax
import jax.numpy as jnp
from jax.experimental import pallas as pl


def kernel(node_feature, edge_index, edge_feature, W_msg, att_msg):
    raise NotImplementedError("write your pallas kernel here")


# speedup vs baseline: 13.4067x; 13.4067x over previous
import functools

import jax
import jax.numpy as jnp
from jax import lax
from jax.experimental import pallas as pl
from jax.experimental.pallas import tpu as pltpu
from jax.experimental.pallas import tpu_sc as plsc

N = 10000
E = 320000
D = 128
ED = 16
H = 4
HC = D // H
NEG_SLOPE = 0.2

HPC = H // 2
DC = D // 2
WIDTH = 80
K = 128
NCHUNK = E // K
NC, NS = 2, 16
NPAD = 10240
ROWS_PER_TILE = NPAD // NS


def _node_body(x_ref, wx_ref, a_ref, xw_ref, ax_ref):
    xw = jnp.dot(x_ref[...], wx_ref[...], preferred_element_type=jnp.float32)
    xw_ref[...] = xw
    ax_ref[...] = jnp.dot(xw, a_ref[...], preferred_element_type=jnp.float32)


def _node_precompute(x, w_x, a_sel):
    blk = 1000
    return pl.pallas_call(
        _node_body,
        grid=(N // blk,),
        in_specs=[
            pl.BlockSpec((blk, D), lambda i: (i, 0)),
            pl.BlockSpec((D, D), lambda i: (0, 0)),
            pl.BlockSpec((D, H), lambda i: (0, 0)),
        ],
        out_specs=[
            pl.BlockSpec((blk, D), lambda i: (i, 0)),
            pl.BlockSpec((blk, H), lambda i: (i, 0)),
        ],
        out_shape=[
            jax.ShapeDtypeStruct((N, D), jnp.float32),
            jax.ShapeDtypeStruct((N, H), jnp.float32),
        ],
    )(x, w_x, a_sel)


def _edge_body(ef_ref, w_ref, out_ref):
    out_ref[...] = jnp.dot(ef_ref[...], w_ref[0],
                           preferred_element_type=jnp.float32)[None]


def _edge_precompute(ef, w_cat):
    blk = 8000
    return pl.pallas_call(
        _edge_body,
        grid=(NC, E // blk),
        in_specs=[
            pl.BlockSpec((blk, ED), lambda c, i: (i, 0)),
            pl.BlockSpec((1, ED, WIDTH), lambda c, i: (c, 0, 0)),
        ],
        out_specs=pl.BlockSpec((1, blk, WIDTH), lambda c, i: (c, i, 0)),
        out_shape=jax.ShapeDtypeStruct((NC, E, WIDTH), jnp.float32),
    )(ef, w_cat)


def _final_body(m_ref, st_ref, out_ref):
    num = jnp.concatenate([m_ref[0, :, :DC], m_ref[1, :, :DC]], axis=-1)
    den4 = jnp.concatenate(
        [m_ref[0, :, DC:DC + HPC], m_ref[1, :, DC:DC + HPC]], axis=-1)
    den = jnp.dot(den4, st_ref[...], preferred_element_type=jnp.float32)
    out_ref[...] = num / (den + 1e-16)


def _finalize(parts, s_t):
    blk = 1000
    return pl.pallas_call(
        _final_body,
        grid=(N // blk,),
        in_specs=[
            pl.BlockSpec((NC, blk, WIDTH), lambda i: (0, i, 0)),
            pl.BlockSpec((H, D), lambda i: (0, 0)),
        ],
        out_specs=pl.BlockSpec((blk, D), lambda i: (i, 0)),
        out_shape=jax.ShapeDtypeStruct((N, D), jnp.float32),
    )(parts, s_t)


def _sc_body(src_hbm, dst_hbm, ew_hbm, xw0_hbm, xw1_hbm, ax_hbm, zero_hbm,
             out_hbm, ax_v, ew_v, rows_v, msg_v, src_v, dst_v, acc_sh, sem):
    cid = lax.axis_index("c")
    sid = lax.axis_index("s")

    pltpu.sync_copy(zero_hbm, acc_sh.at[pl.ds(sid * ROWS_PER_TILE, ROWS_PER_TILE)])
    pltpu.sync_copy(zero_hbm.at[pl.ds(0, K)], msg_v)
    pltpu.sync_copy(ax_hbm.at[cid], ax_v)
    plsc.subcore_barrier()

    rem = NCHUNK - (NCHUNK // NS) * NS
    n_i = (NCHUNK // NS) + jnp.where(sid < rem, 1, 0)

    def chunk_body(i, carry):
        ci = sid + i * NS
        base = ci * K
        pltpu.sync_copy(src_hbm.at[pl.ds(base, K)], src_v)
        pltpu.sync_copy(dst_hbm.at[pl.ds(base, K)], dst_v)
        pltpu.sync_copy(ew_hbm.at[cid, pl.ds(base, K)], ew_v)
        @pl.when(cid == 0)
        def _gather0():
            pltpu.async_copy(xw0_hbm.at[src_v], rows_v, sem).wait()

        @pl.when(cid == 1)
        def _gather1():
            pltpu.async_copy(xw1_hbm.at[src_v], rows_v, sem).wait()

        for b in range(K // 16):
            r_lanes = lax.iota(jnp.int32, 16) + (16 * b)
            src_vec = src_v[pl.ds(16 * b, 16)]
            p_list = []
            for j in range(HPC):
                col_j = jnp.full((16,), DC + j, jnp.int32)
                ax_g = plsc.load_gather(ax_v, [src_vec * HPC + j])
                ae_g = plsc.load_gather(ew_v, [r_lanes, col_j])
                al = ax_g + ae_g
                al = jnp.where(al >= 0.0, al, NEG_SLOPE * al)
                p = jnp.exp(al)
                plsc.store_scatter(msg_v, [r_lanes, col_j], p)
                p_list.append(p)
            for j in range(HPC):
                p = p_list[j]

                def ch_body(c2, _, p=p, r_lanes=r_lanes, j=j):
                    col = jnp.full((16,), 1, jnp.int32) * (HC * j + c2)
                    g = plsc.load_gather(rows_v, [r_lanes, col])
                    w = plsc.load_gather(ew_v, [r_lanes, col])
                    plsc.store_scatter(msg_v, [r_lanes, col], (g + w) * p)
                    return _

                lax.fori_loop(0, HC, ch_body, None, unroll=4)

        pltpu.sync_copy(msg_v, acc_sh.at[dst_v], add=True)
        return carry

    lax.fori_loop(0, n_i, chunk_body, None)

    plsc.subcore_barrier()
    pltpu.sync_copy(acc_sh.at[pl.ds(sid * ROWS_PER_TILE, ROWS_PER_TILE)],
                    out_hbm.at[cid, pl.ds(sid * ROWS_PER_TILE, ROWS_PER_TILE)])


_sc_main = functools.partial(
    pl.kernel,
    out_type=jax.ShapeDtypeStruct((NC, NPAD, WIDTH), jnp.float32),
    mesh=plsc.VectorSubcoreMesh(core_axis_name="c", subcore_axis_name="s"),
    compiler_params=pltpu.CompilerParams(needs_layout_passes=False,
                                         use_tc_tiling_on_sc=False),
    scratch_types=[
        pltpu.VMEM((N * HPC,), jnp.float32),
        pltpu.VMEM((K, WIDTH), jnp.float32),
        pltpu.VMEM((K, DC), jnp.float32),
        pltpu.VMEM((K, WIDTH), jnp.float32),
        pltpu.VMEM((K,), jnp.int32),
        pltpu.VMEM((K,), jnp.int32),
        pltpu.VMEM_SHARED((NPAD, WIDTH), jnp.float32),
        pltpu.SemaphoreType.DMA,
    ],
)(_sc_body)


def kernel(node_feature, edge_index, edge_feature, W_msg, att_msg):
    src = edge_index[0].astype(jnp.int32)
    dst = edge_index[1].astype(jnp.int32)
    w_x = W_msg[:D]
    w_e = W_msg[D:]

    att_flat = att_msg.reshape(D)
    sel = (jnp.arange(D, dtype=jnp.int32)[:, None] // HC
           == jnp.arange(H, dtype=jnp.int32)[None, :]).astype(jnp.float32)
    a_sel = att_flat[:, None] * sel
    v = w_e @ a_sel
    pad = jnp.zeros((ED, WIDTH - DC - HPC), jnp.float32)
    w_cat = jnp.stack([
        jnp.concatenate([w_e[:, :DC], v[:, :HPC], pad], axis=1),
        jnp.concatenate([w_e[:, DC:], v[:, HPC:], pad], axis=1),
    ])

    xw, ax = _node_precompute(node_feature, w_x, a_sel)
    ew = _edge_precompute(edge_feature, w_cat)

    xw0 = xw[:, :DC]
    xw1 = xw[:, DC:]
    ax_r = ax.reshape(N, NC, HPC).transpose(1, 0, 2).reshape(NC, N * HPC)
    zeros_buf = jnp.zeros((ROWS_PER_TILE, WIDTH), jnp.float32)

    parts = _sc_main(src, dst, ew, xw0, xw1, ax_r, zeros_buf)

    return _finalize(parts[:, :N], sel.T)
